# Initial kernel scaffold; baseline (speedup 1.0000x reference)
import jax
import jax.numpy as jnp
from jax.experimental import pallas as pl


def kernel(x, som):
    raise NotImplementedError("write your pallas kernel here")


# speedup vs baseline: 6.6362x; 6.6362x over previous
import functools

import jax
import jax.numpy as jnp
from jax.experimental import pallas as pl

H, W, C, KH, KW = 32, 32, 32, 4, 4
B, XH, XW = 16, 16, 16
D = C * KH * KW
NQ = H * W
NPOS = B * XH * XW
NR = XH - KH + 1


def _som_kernel(xt_ref, codes_ref, out_ref):
    codes = codes_ref[...]
    c2 = jnp.sum(codes * codes, axis=0)
    parts = [
        xt_ref[pl.ds(kh * XW + kw, NPOS), :]
        for kh in range(KH)
        for kw in range(KW)
    ]
    p = jnp.concatenate(parts, axis=1)
    p2 = jnp.sum(p * p, axis=1, keepdims=True)
    cross = jax.lax.dot_general(
        p, codes, (((1,), (0,)), ((), ())),
        preferred_element_type=jnp.float32)
    mse = (p2 - 2.0 * cross + c2[None, :]) / D
    idx = jnp.argmin(mse, axis=1)
    wr = (idx // W).astype(jnp.float32) / H
    wc = (idx % W).astype(jnp.float32) / W
    out_ref[...] = jnp.stack([wr, wc], axis=0)


@jax.jit
def kernel(x, som):
    xt = x.transpose(0, 2, 3, 1).reshape(NPOS, C)
    xt = jnp.pad(xt, ((0, KH * XW), (0, 0)))
    codes = som.transpose(3, 4, 2, 0, 1).reshape(D, NQ)
    out = pl.pallas_call(
        _som_kernel,
        out_shape=jax.ShapeDtypeStruct((2, NPOS), jnp.float32),
    )(xt, codes)
    out = out.reshape(2, B, XH, XW)[:, :, :NR, :NR]
    return out.transpose(1, 0, 2, 3)
